# baseline (speedup 1.0000x reference)
import jax
import jax.numpy as jnp
from jax.experimental import pallas as pl
from jax.experimental.pallas import tpu as pltpu

H = 64


def _encoder_body(idx_ref, table_ref, h0_ref, c0_ref,
                  wih0_ref, whh0_ref, b_ih0_ref, b_hh0_ref,
                  wih1_ref, whh1_ref, b_ih1_ref, b_hh1_ref,
                  out_ref, h_ref, c_ref):
    sub = idx_ref[0] % 8
    x = table_ref[pl.ds(sub, 1), :]

    def cell(xv, hv, cv, wih, whh, b_ih, b_hh):
        dn = (((1,), (1,)), ((), ()))
        gates = (jax.lax.dot_general(xv, wih, dn, preferred_element_type=jnp.float32)
                 + jax.lax.dot_general(hv, whh, dn, preferred_element_type=jnp.float32)
                 + b_ih + b_hh)
        ig = jax.nn.sigmoid(gates[:, 0:H])
        fg = jax.nn.sigmoid(gates[:, H:2 * H])
        gg = jnp.tanh(gates[:, 2 * H:3 * H])
        og = jax.nn.sigmoid(gates[:, 3 * H:4 * H])
        c_new = fg * cv + ig * gg
        h_new = og * jnp.tanh(c_new)
        return h_new, c_new

    h1, c1 = cell(x, h0_ref[0:1, :], c0_ref[0:1, :],
                  wih0_ref[...], whh0_ref[...], b_ih0_ref[...], b_hh0_ref[...])
    h2, c2 = cell(h1, h0_ref[1:2, :], c0_ref[1:2, :],
                  wih1_ref[...], whh1_ref[...], b_ih1_ref[...], b_hh1_ref[...])

    out_ref[...] = h2
    h_ref[0:1, :] = h1
    h_ref[1:2, :] = h2
    c_ref[0:1, :] = c1
    c_ref[1:2, :] = c2


def kernel(input, h0, c0, table, W_ih0, W_hh0, b_ih0, b_hh0, W_ih1, W_hh1, b_ih1, b_hh1):
    f32 = jnp.float32
    full = lambda shape: pl.BlockSpec(shape, lambda i, idx: (0,) * len(shape))
    grid_spec = pltpu.PrefetchScalarGridSpec(
        num_scalar_prefetch=1,
        grid=(1,),
        in_specs=[
            pl.BlockSpec((8, H), lambda i, idx: (idx[0] // 8, 0)),
            full((2, H)),
            full((2, H)),
            full((4 * H, H)),
            full((4 * H, H)),
            full((1, 4 * H)),
            full((1, 4 * H)),
            full((4 * H, H)),
            full((4 * H, H)),
            full((1, 4 * H)),
            full((1, 4 * H)),
        ],
        out_specs=[
            full((1, H)),
            full((2, H)),
            full((2, H)),
        ],
    )
    out, h_new, c_new = pl.pallas_call(
        _encoder_body,
        grid_spec=grid_spec,
        out_shape=[
            jax.ShapeDtypeStruct((1, H), f32),
            jax.ShapeDtypeStruct((2, H), f32),
            jax.ShapeDtypeStruct((2, H), f32),
        ],
    )(
        input, table,
        h0.reshape(2, H), c0.reshape(2, H),
        W_ih0, W_hh0, b_ih0.reshape(1, 4 * H), b_hh0.reshape(1, 4 * H),
        W_ih1, W_hh1, b_ih1.reshape(1, 4 * H), b_hh1.reshape(1, 4 * H),
    )
    return (out.reshape(1, 1, H), h_new.reshape(2, 1, H), c_new.reshape(2, 1, H))


# speedup vs baseline: 104.2925x; 104.2925x over previous
import jax
import jax.numpy as jnp
from jax.experimental import pallas as pl
from jax.experimental.pallas import tpu as pltpu

H = 64
LANES = 128


def _encoder_body(idx_ref, tcol_ref, h0_ref, c0_ref,
                  wih0_ref, whh0_ref, b_ih0_ref, b_hh0_ref,
                  wih1_ref, whh1_ref, b_ih1_ref, b_hh1_ref,
                  out_ref, h_ref, c_ref):
    col = idx_ref[0] % LANES
    lane = jax.lax.broadcasted_iota(jnp.int32, (H, LANES), 1)
    x_col = jnp.sum(jnp.where(lane == col, tcol_ref[...], 0.0),
                    axis=1, keepdims=True)

    b0 = b_ih0_ref[...].reshape(1, 4 * H) + b_hh0_ref[...].reshape(1, 4 * H)
    b1 = b_ih1_ref[...].reshape(1, 4 * H) + b_hh1_ref[...].reshape(1, 4 * H)

    def gates_to_state(gates, cv):
        ig = jax.nn.sigmoid(gates[:, 0:H])
        fg = jax.nn.sigmoid(gates[:, H:2 * H])
        gg = jnp.tanh(gates[:, 2 * H:3 * H])
        og = jax.nn.sigmoid(gates[:, 3 * H:4 * H])
        c_new = fg * cv + ig * gg
        h_new = og * jnp.tanh(c_new)
        return h_new, c_new

    gates0 = (jnp.sum(x_col * wih0_ref[...], axis=0, keepdims=True)
              + jnp.dot(h0_ref[0], whh0_ref[...], preferred_element_type=jnp.float32)
              + b0)
    h1, c1 = gates_to_state(gates0, c0_ref[0])

    gates1 = (jnp.dot(h1, wih1_ref[...], preferred_element_type=jnp.float32)
              + jnp.dot(h0_ref[1], whh1_ref[...], preferred_element_type=jnp.float32)
              + b1)
    h2, c2 = gates_to_state(gates1, c0_ref[1])

    out_ref[0] = h2
    h_ref[0] = h1
    h_ref[1] = h2
    c_ref[0] = c1
    c_ref[1] = c2


def kernel(input, h0, c0, table, W_ih0, W_hh0, b_ih0, b_hh0, W_ih1, W_hh1, b_ih1, b_hh1):
    f32 = jnp.float32
    full = lambda shape: pl.BlockSpec(shape, lambda i, idx: (0,) * len(shape))
    grid_spec = pltpu.PrefetchScalarGridSpec(
        num_scalar_prefetch=1,
        grid=(1,),
        in_specs=[
            pl.BlockSpec((H, LANES), lambda i, idx: (0, idx[0] // LANES)),
            full((2, 1, H)),
            full((2, 1, H)),
            full((H, 4 * H)),
            full((H, 4 * H)),
            full((4 * H,)),
            full((4 * H,)),
            full((H, 4 * H)),
            full((H, 4 * H)),
            full((4 * H,)),
            full((4 * H,)),
        ],
        out_specs=[
            full((1, 1, H)),
            full((2, 1, H)),
            full((2, 1, H)),
        ],
    )
    return tuple(pl.pallas_call(
        _encoder_body,
        grid_spec=grid_spec,
        out_shape=[
            jax.ShapeDtypeStruct((1, 1, H), f32),
            jax.ShapeDtypeStruct((2, 1, H), f32),
            jax.ShapeDtypeStruct((2, 1, H), f32),
        ],
    )(
        input, table.T,
        h0, c0,
        W_ih0.T, W_hh0.T, b_ih0, b_hh0,
        W_ih1.T, W_hh1.T, b_ih1, b_hh1,
    ))
